# baseline (speedup 1.0000x reference)
import functools

import jax
import jax.numpy as jnp
from jax.experimental import pallas as pl
from jax.experimental.pallas import tpu as pltpu

N_CHANNELS = 4096
N_ENERGIES = 8192
BC = 128


def _matvec_kernel(params_ref, energies_ref, tm_ref, out_ref, flux_ref):
    @pl.when(pl.program_id(0) == 0)
    def _flux():
        alpha = params_ref[0, 0]
        norm = params_ref[0, 1]
        oma = 1.0 - alpha
        e_lo = energies_ref[0, :]
        e_hi = energies_ref[1, :]
        flux_ref[...] = (
            (norm / oma)
            * (jnp.exp(oma * jnp.log(e_hi)) - jnp.exp(oma * jnp.log(e_lo)))
        ).reshape(N_ENERGIES, 1)

    res = jnp.dot(tm_ref[...], flux_ref[...], preferred_element_type=jnp.float32)
    out_ref[...] = jnp.maximum(res, 1e-6)


def kernel(parameters, energies, transfer_matrix):
    params2d = parameters.reshape(1, 2)
    grid = N_CHANNELS // BC
    out = pl.pallas_call(
        _matvec_kernel,
        grid=(grid,),
        in_specs=[
            pl.BlockSpec((1, 2), lambda i: (0, 0), memory_space=pltpu.SMEM),
            pl.BlockSpec((2, N_ENERGIES), lambda i: (0, 0)),
            pl.BlockSpec((BC, N_ENERGIES), lambda i: (i, 0)),
        ],
        out_specs=pl.BlockSpec((BC, 1), lambda i: (i, 0)),
        out_shape=jax.ShapeDtypeStruct((N_CHANNELS, 1), jnp.float32),
        scratch_shapes=[pltpu.VMEM((N_ENERGIES, 1), jnp.float32)],
    )(params2d, energies, transfer_matrix)
    return out.reshape(N_CHANNELS)


# speedup vs baseline: 1.1917x; 1.1917x over previous
import functools

import jax
import jax.numpy as jnp
from jax.experimental import pallas as pl
from jax.experimental.pallas import tpu as pltpu

N_CHANNELS = 4096
N_ENERGIES = 8192
BC = 256


def _matvec_kernel(params_ref, energies_ref, tm_ref, out_ref, flux_ref):
    @pl.when(pl.program_id(0) == 0)
    def _flux():
        alpha = params_ref[0, 0]
        norm = params_ref[0, 1]
        oma = 1.0 - alpha
        e_lo = energies_ref[0, :]
        e_hi = energies_ref[1, :]
        flux_ref[...] = (
            (norm / oma)
            * (jnp.exp(oma * jnp.log(e_hi)) - jnp.exp(oma * jnp.log(e_lo)))
        ).reshape(N_ENERGIES, 1)

    res = jnp.dot(tm_ref[...], flux_ref[...], preferred_element_type=jnp.float32)
    out_ref[...] = jnp.maximum(res, 1e-6)


def kernel(parameters, energies, transfer_matrix):
    params2d = parameters.reshape(1, 2)
    grid = N_CHANNELS // BC
    out = pl.pallas_call(
        _matvec_kernel,
        grid=(grid,),
        in_specs=[
            pl.BlockSpec((1, 2), lambda i: (0, 0), memory_space=pltpu.SMEM),
            pl.BlockSpec((2, N_ENERGIES), lambda i: (0, 0)),
            pl.BlockSpec((BC, N_ENERGIES), lambda i: (i, 0)),
        ],
        out_specs=pl.BlockSpec((BC, 1), lambda i: (i, 0)),
        out_shape=jax.ShapeDtypeStruct((N_CHANNELS, 1), jnp.float32),
        scratch_shapes=[pltpu.VMEM((N_ENERGIES, 1), jnp.float32)],
    )(params2d, energies, transfer_matrix)
    return out.reshape(N_CHANNELS)
